# baseline (speedup 1.0000x reference)
import functools

import jax
import jax.numpy as jnp
from jax.experimental import pallas as pl

_F32 = jnp.float32


def _dot(a, b):
    return jax.lax.dot_general(a, b, (((1,), (0,)), ((), ())),
                               preferred_element_type=_F32)


def _interleave(a, b):
    C, S = a.shape
    return jnp.stack([a, b], axis=-1).reshape(C, 2 * S)


def _colshift(v, W):
    C, S = v.shape
    sh = jnp.concatenate([jnp.zeros((C, 1), _F32), v[:, :S - 1]], axis=1)
    lane = jax.lax.broadcasted_iota(jnp.int32, (1, S), 1)
    return jnp.where(lane % W == 0, jnp.zeros((), _F32), sh)


def _head(x, mask, mwt1, mb1, mwt2, mb2, owt, ob, with_mpred):
    if with_mpred:
        h = jax.nn.relu(_dot(mwt1, x) + mb1) * mask
        logit = (_dot(mwt2, h) + mb2) * mask
        this_mask = mask * (logit[1:2] > logit[0:1]).astype(_F32)
    else:
        logit = None
        this_mask = mask
    out = jax.nn.sigmoid(_dot(owt, x) + ob) * this_mask
    return out, logit, this_mask


def _level0_body(x_ref, m_ref, mwt1_ref, mb1_ref, mwt2_ref, mb2_ref,
                 owt_ref, ob_ref, out_ref, logit_ref, tm_ref, mres_ref):
    x = x_ref[0]
    mask = m_ref[0]
    out, logit, this_mask = _head(x, mask, mwt1_ref[...], mb1_ref[...],
                                  mwt2_ref[...], mb2_ref[...],
                                  owt_ref[...], ob_ref[...], True)
    out_ref[0] = out
    logit_ref[0] = logit
    tm_ref[0] = this_mask
    mres_ref[0] = mask - this_mask


def _level0_kernel(x, mask, mw1, mb1, mw2, mb2, ow, ob):
    B, C, S = x.shape
    bs = lambda c: pl.BlockSpec((1, c, S), lambda b: (b, 0, 0))
    full = lambda a: pl.BlockSpec(a.shape, lambda b: (0,) * a.ndim)
    args = (x, mask, mw1.T, mb1[:, None], mw2.T, mb2[:, None],
            ow.T, ob[:, None])
    in_specs = [bs(C), bs(1)] + [full(a) for a in args[2:]]
    out_shape = [jax.ShapeDtypeStruct((B, c, S), _F32) for c in (1, 2, 1, 1)]
    out_specs = [bs(1), bs(2), bs(1), bs(1)]
    fn = pl.pallas_call(_level0_body, grid=(B,), in_specs=in_specs,
                        out_specs=out_specs, out_shape=out_shape)
    return fn(*args)


def _level_body(with_mpred, want_x, Co, W, rb,
                x_ref, xh_ref, m_ref, skip_ref, wall_ref, b_ref, *rest):
    if with_mpred:
        (mwt1_ref, mb1_ref, mwt2_ref, mb2_ref, owt_ref, ob_ref,
         xout_ref, out_ref, logit_ref, tm_ref, mres_ref) = rest
    else:
        mwt1_ref = mb1_ref = mwt2_ref = mb2_ref = None
        owt_ref, ob_ref, out_ref, tm_ref = rest
        xout_ref = logit_ref = mres_ref = None

    i = pl.program_id(1)
    first = jnp.where(i == 0, 0.0, 1.0).astype(_F32)
    x = x_ref[0]
    xh = xh_ref[0, :, 0, 0]
    wall = wall_ref[...]
    con = _dot(wall, x)
    conh = _dot(wall[:3 * Co], xh) * first

    S = rb * W
    c = lambda t: con[t * Co:(t + 1) * Co]
    ch = lambda t: conh[t * Co:(t + 1) * Co]
    rs = lambda t: jnp.concatenate([ch(t), c(t)[:, :S - W]], axis=1)
    c00, c01, c02 = rs(0), rs(1), rs(2)
    c10, c11, c12 = c(3), c(4), c(5)
    c20, c21, c22 = c(6), c(7), c(8)
    pe = c22 + c02 + _colshift(c20 + c00, W)
    po = c21 + c01
    qe = c12 + _colshift(c10, W)
    qo = c11
    ye = _interleave(pe, po)
    yo = _interleave(qe, qo)

    m = m_ref[0]
    mc = _interleave(m, m)

    W2 = 2 * W
    xr, mr = [], []
    for r in range(rb):
        sl = slice(r * W2, (r + 1) * W2)
        xr += [ye[:, sl], yo[:, sl]]
        mr += [mc[:, sl], mc[:, sl]]
    y = jnp.concatenate(xr, axis=1) + b_ref[...]
    mask = jnp.concatenate(mr, axis=1)
    skip = skip_ref[0]
    xv = (jnp.where(y >= 0, y, 0.01 * y) * mask + skip * mask) * 0.5
    if want_x:
        xout_ref[0] = xv
    out, logit, this_mask = _head(
        xv, mask,
        None if mwt1_ref is None else mwt1_ref[...],
        None if mb1_ref is None else mb1_ref[...],
        None if mwt2_ref is None else mwt2_ref[...],
        None if mb2_ref is None else mb2_ref[...],
        owt_ref[...], ob_ref[...], with_mpred)
    out_ref[0] = out
    tm_ref[0] = this_mask
    if with_mpred:
        logit_ref[0] = logit
        mres_ref[0] = mask - this_mask


def _level_kernel(x, mask, skip, dw, db, mw1, mb1, mw2, mb2, ow, ob,
                  H, W, rb, with_mpred, want_x):
    B, Cin, S_full = x.shape
    Co = dw.shape[3]
    grid = (B, H // rb)
    S = rb * W
    S2 = 4 * S
    wall = dw.transpose(0, 1, 3, 2).reshape(9 * Co, Cin)

    xin = pl.BlockSpec((1, Cin, S), lambda b, i: (b, 0, i))
    xh = pl.BlockSpec((1, Cin, 1, 1, W),
                      lambda b, i: (b, 0, jnp.maximum(i * rb - 1, 0), 0, 0))
    min_ = pl.BlockSpec((1, 1, S), lambda b, i: (b, 0, i))
    sin = pl.BlockSpec((1, Co, S2), lambda b, i: (b, 0, i))
    full = lambda a: pl.BlockSpec(a.shape, lambda b, i: (0,) * a.ndim)
    ob2 = lambda cdim: pl.BlockSpec((1, cdim, S2), lambda b, i: (b, 0, i))

    args = [x, x.reshape(B, Cin, H, 1, W), mask, skip, wall, db[:, None]]
    in_specs = [xin, xh, min_, sin, full(wall), full(args[-1])]
    if with_mpred:
        args += [mw1.T, mb1[:, None], mw2.T, mb2[:, None]]
        in_specs += [full(a) for a in args[-4:]]
    args += [ow.T, ob[:, None]]
    in_specs += [full(args[-2]), full(args[-1])]

    S2_full = 4 * S_full
    out_shape, out_specs = [], []
    if want_x:
        out_shape.append(jax.ShapeDtypeStruct((B, Co, S2_full), _F32))
        out_specs.append(ob2(Co))
    out_shape.append(jax.ShapeDtypeStruct((B, 1, S2_full), _F32))
    out_specs.append(ob2(1))
    if with_mpred:
        out_shape.append(jax.ShapeDtypeStruct((B, 2, S2_full), _F32))
        out_specs.append(ob2(2))
    out_shape.append(jax.ShapeDtypeStruct((B, 1, S2_full), _F32))
    out_specs.append(ob2(1))
    if with_mpred:
        out_shape.append(jax.ShapeDtypeStruct((B, 1, S2_full), _F32))
        out_specs.append(ob2(1))

    fn = pl.pallas_call(
        functools.partial(_level_body, with_mpred, want_x, Co, W, rb),
        grid=grid, in_specs=in_specs, out_specs=out_specs,
        out_shape=out_shape)
    return fn(*args)


def kernel(input, imask, skip_1, skip_2, skip_3, dw1, db1, dw2, db2, dw3,
           db3, ow0, ob0, ow1, ob1, ow2, ob2, ow3, ob3, mw1_0, mb1_0,
           mw2_0, mb2_0, mw1_1, mb1_1, mw2_1, mb2_1, mw1_2, mb1_2, mw2_2,
           mb2_2):
    B = input.shape[0]
    fl = lambda a: a.reshape(a.shape[0], a.shape[1], -1)
    out0, l0, tm0, mres0 = _level0_kernel(
        fl(input), fl(imask), mw1_0, mb1_0, mw2_0, mb2_0, ow0, ob0)
    x1, out1, l1, tm1, mres1 = _level_kernel(
        fl(input), mres0, fl(skip_1), dw1, db1, mw1_1, mb1_1, mw2_1,
        mb2_1, ow1, ob1, H=32, W=32, rb=8, with_mpred=True, want_x=True)
    x2, out2, l2, tm2, mres2 = _level_kernel(
        x1, mres1, fl(skip_2), dw2, db2, mw1_2, mb1_2, mw2_2, mb2_2,
        ow2, ob2, H=64, W=64, rb=8, with_mpred=True, want_x=True)
    out3, tm3 = _level_kernel(
        x2, mres2, fl(skip_3), dw3, db3, None, None, None, None,
        ow3, ob3, H=128, W=128, rb=8, with_mpred=False, want_x=False)
    sq = lambda a, c, h: a.reshape(B, c, h, h)
    return (sq(out0, 1, 32), sq(out1, 1, 64), sq(out2, 1, 128),
            sq(out3, 1, 256), sq(l0, 2, 32), sq(l1, 2, 64),
            sq(l2, 2, 128), sq(tm0, 1, 32), sq(tm1, 1, 64),
            sq(tm2, 1, 128), sq(tm3, 1, 256))


# speedup vs baseline: 5.6655x; 5.6655x over previous
import functools

import jax
import jax.numpy as jnp
from jax.experimental import pallas as pl

_F32 = jnp.float32
_PL = 1024


def _dot(a, b):
    return jax.lax.dot_general(a, b, (((1,), (0,)), ((), ())),
                               preferred_element_type=_F32)


def _rollrows(v):
    C = v.shape[0]
    return jnp.concatenate([jnp.zeros((C, 32), _F32), v[:, :_PL - 32]],
                           axis=1)


def _rollcols(v):
    C = v.shape[0]
    sh = jnp.concatenate([jnp.zeros((C, 1), _F32), v[:, :_PL - 1]], axis=1)
    lane = jax.lax.broadcasted_iota(jnp.int32, (1, _PL), 1)
    return jnp.where(lane % 32 == 0, jnp.zeros((), _F32), sh)


def _head(x, mask, mwt1, mb1, mwt2, mb2, owt, ob, with_mpred):
    if with_mpred:
        h = jax.nn.relu(_dot(mwt1, x) + mb1) * mask
        logit = (_dot(mwt2, h) + mb2) * mask
        this_mask = mask * (logit[1:2] > logit[0:1]).astype(_F32)
    else:
        logit = None
        this_mask = mask
    out = jax.nn.sigmoid(_dot(owt, x) + ob) * this_mask
    return out, logit, this_mask


def _level0_body(x_ref, m_ref, mwt1_ref, mb1_ref, mwt2_ref, mb2_ref,
                 owt_ref, ob_ref, out_ref, logit_ref, tm_ref, mres_ref):
    x = x_ref[0]
    mask = m_ref[0]
    out, logit, this_mask = _head(x, mask, mwt1_ref[...], mb1_ref[...],
                                  mwt2_ref[...], mb2_ref[...],
                                  owt_ref[...], ob_ref[...], True)
    out_ref[0] = out
    logit_ref[0] = logit
    tm_ref[0] = this_mask
    mres_ref[0] = mask - this_mask


def _level0_kernel(x, mask, mw1, mb1, mw2, mb2, ow, ob):
    B, C, S = x.shape
    bs = lambda c: pl.BlockSpec((1, c, S), lambda b: (b, 0, 0))
    full = lambda a: pl.BlockSpec(a.shape, lambda b: (0,) * a.ndim)
    args = (x, mask, mw1.T, mb1[:, None], mw2.T, mb2[:, None],
            ow.T, ob[:, None])
    in_specs = [bs(C), bs(1)] + [full(a) for a in args[2:]]
    out_shape = [jax.ShapeDtypeStruct((B, c, S), _F32) for c in (1, 2, 1, 1)]
    out_specs = [bs(1), bs(2), bs(1), bs(1)]
    fn = pl.pallas_call(_level0_body, grid=(B,), in_specs=in_specs,
                        out_specs=out_specs, out_shape=out_shape)
    return fn(*args)


def _level_body(with_mpred, want_x, Co,
                x_ref, xu_ref, xl_ref, xul_ref, m_ref, skip_ref,
                wa_ref, wb_ref, wc_ref, wd_ref, b_ref, *rest):
    if with_mpred:
        (mwt1_ref, mb1_ref, mwt2_ref, mb2_ref, owt_ref, ob_ref,
         xout_ref, out_ref, logit_ref, tm_ref, mres_ref) = rest
    else:
        mwt1_ref = mb1_ref = mwt2_ref = mb2_ref = None
        owt_ref, ob_ref, out_ref, tm_ref = rest
        xout_ref = logit_ref = mres_ref = None

    S = pl.program_id(1)
    T = pl.program_id(2)
    pv = lambda r: r[0, :, 0, 0, 0, :]
    P = pv(x_ref)
    Pu_r, Pl_r, Pul_r = pv(xu_ref), pv(xl_ref), pv(xul_ref)
    Pu = jnp.where(S == 0, _rollrows(Pu_r), Pu_r)
    Pl = jnp.where(T == 0, _rollcols(Pl_r), Pl_r)
    Pul = jnp.where(S == 0, _rollrows(Pul_r), Pul_r)
    Pul = jnp.where(T == 0, _rollcols(Pul), Pul)

    A = _dot(wa_ref[...], P)
    Bv = _dot(wb_ref[...], Pu)
    Cv = _dot(wc_ref[...], Pl)
    Dv = _dot(wd_ref[...], Pul)
    phases = [
        A[:Co] + Bv[:Co] + Cv[:Co] + Dv,
        A[Co:2 * Co] + Bv[Co:2 * Co],
        A[2 * Co:3 * Co] + Cv[Co:2 * Co],
        A[3 * Co:4 * Co],
    ]

    m = m_ref[0, :, 0, 0, 0, :]
    skip = skip_ref[0, :, 0, 0, 0, :]
    db = b_ref[...]
    for k in range(4):
        sl = slice(k * _PL, (k + 1) * _PL)
        y = phases[k] + db
        xk = (jnp.where(y >= 0, y, 0.01 * y) * m + skip[:, sl] * m) * 0.5
        if want_x:
            xout_ref[0, :, 0, 0, 0, sl] = xk
        out, logit, this_mask = _head(
            xk, m,
            None if mwt1_ref is None else mwt1_ref[...],
            None if mb1_ref is None else mb1_ref[...],
            None if mwt2_ref is None else mwt2_ref[...],
            None if mb2_ref is None else mb2_ref[...],
            owt_ref[...], ob_ref[...], with_mpred)
        out_ref[0, :, 0, 0, 0, sl] = out
        tm_ref[0, :, 0, 0, 0, sl] = this_mask
        if with_mpred:
            logit_ref[0, :, 0, 0, 0, sl] = logit
            mres_ref[0, :, 0, 0, 0, sl] = m - this_mask


def _level_kernel(x, mask, skip, dw, db, mw1, mb1, mw2, mb2, ow, ob,
                  G, with_mpred, want_x):
    B, Cin = x.shape[0], x.shape[1]
    Co = dw.shape[3]
    grid = (B, G, G)
    wt = lambda kh, kw: dw[kh, kw].T
    wa = jnp.concatenate([wt(2, 2), wt(2, 1), wt(1, 2), wt(1, 1)], axis=0)
    wb = jnp.concatenate([wt(0, 2), wt(0, 1)], axis=0)
    wc = jnp.concatenate([wt(2, 0), wt(1, 0)], axis=0)
    wd = wt(0, 0)

    def pidx(ds_, dt):
        def f(b, s, t):
            s2 = (s - ds_) % G
            t2 = (t - dt) % G
            return (b, 0, s2 // 2, t2 // 2, 0, (s2 % 2) * 2 + (t2 % 2))
        return f

    xbs = lambda c: pl.BlockSpec((1, c, 1, 1, 1, _PL), pidx(0, 0))
    obs = lambda c: pl.BlockSpec((1, c, 1, 1, 1, 4 * _PL),
                                 lambda b, s, t: (b, 0, s, t, 0, 0))
    full = lambda a: pl.BlockSpec(a.shape, lambda b, s, t: (0,) * a.ndim)

    args = [x, x, x, x, mask, skip, wa, wb, wc, wd, db[:, None]]
    in_specs = [xbs(Cin),
                pl.BlockSpec((1, Cin, 1, 1, 1, _PL), pidx(1, 0)),
                pl.BlockSpec((1, Cin, 1, 1, 1, _PL), pidx(0, 1)),
                pl.BlockSpec((1, Cin, 1, 1, 1, _PL), pidx(1, 1)),
                pl.BlockSpec((1, 1, 1, 1, 1, _PL), pidx(0, 0)),
                obs(Co), full(wa), full(wb), full(wc), full(wd),
                full(args[-1])]
    if with_mpred:
        args += [mw1.T, mb1[:, None], mw2.T, mb2[:, None]]
        in_specs += [full(a) for a in args[-4:]]
    args += [ow.T, ob[:, None]]
    in_specs += [full(args[-2]), full(args[-1])]

    oshape = lambda c: jax.ShapeDtypeStruct((B, c, G, G, 1, 4 * _PL), _F32)
    out_shape, out_specs = [], []
    if want_x:
        out_shape.append(oshape(Co))
        out_specs.append(obs(Co))
    out_shape.append(oshape(1))
    out_specs.append(obs(1))
    if with_mpred:
        out_shape.append(oshape(2))
        out_specs.append(obs(2))
    out_shape.append(oshape(1))
    out_specs.append(obs(1))
    if with_mpred:
        out_shape.append(oshape(1))
        out_specs.append(obs(1))

    fn = pl.pallas_call(
        functools.partial(_level_body, with_mpred, want_x, Co),
        grid=grid, in_specs=in_specs, out_specs=out_specs,
        out_shape=out_shape)
    return fn(*args)


def _to_planes(a, G):
    B, C = a.shape[0], a.shape[1]
    a = a.reshape(B, C, 32, G, 2, 32, G, 2)
    a = a.transpose(0, 1, 3, 6, 4, 7, 2, 5)
    return a.reshape(B, C, G, G, 1, 4 * _PL)


def _from_planes(a, G):
    B, C = a.shape[0], a.shape[1]
    a = a.reshape(B, C, G, G, 2, 2, 32, 32)
    a = a.transpose(0, 1, 6, 2, 4, 7, 3, 5)
    return a.reshape(B, C, 64 * G, 64 * G)


def kernel(input, imask, skip_1, skip_2, skip_3, dw1, db1, dw2, db2, dw3,
           db3, ow0, ob0, ow1, ob1, ow2, ob2, ow3, ob3, mw1_0, mb1_0,
           mw2_0, mb2_0, mw1_1, mb1_1, mw2_1, mb2_1, mw1_2, mb1_2, mw2_2,
           mb2_2):
    B = input.shape[0]
    out0, l0, tm0, mres0 = _level0_kernel(
        input.reshape(B, 256, _PL), imask.reshape(B, 1, _PL),
        mw1_0, mb1_0, mw2_0, mb2_0, ow0, ob0)
    x1, out1, l1, tm1, mres1 = _level_kernel(
        input.reshape(B, 256, 1, 1, 1, _PL),
        mres0.reshape(B, 1, 1, 1, 1, _PL), _to_planes(skip_1, 1),
        dw1, db1, mw1_1, mb1_1, mw2_1, mb2_1, ow1, ob1,
        G=1, with_mpred=True, want_x=True)
    x2, out2, l2, tm2, mres2 = _level_kernel(
        x1, mres1, _to_planes(skip_2, 2), dw2, db2, mw1_2, mb1_2,
        mw2_2, mb2_2, ow2, ob2, G=2, with_mpred=True, want_x=True)
    out3, tm3 = _level_kernel(
        x2, mres2, _to_planes(skip_3, 4), dw3, db3, None, None, None,
        None, ow3, ob3, G=4, with_mpred=False, want_x=False)
    sq = lambda a, c: a.reshape(B, c, 32, 32)
    return (sq(out0, 1), _from_planes(out1, 1), _from_planes(out2, 2),
            _from_planes(out3, 4), sq(l0, 2), _from_planes(l1, 1),
            _from_planes(l2, 2), sq(tm0, 1), _from_planes(tm1, 1),
            _from_planes(tm2, 2), _from_planes(tm3, 4))


# speedup vs baseline: 6.9783x; 1.2317x over previous
import functools

import jax
import jax.numpy as jnp
from jax.experimental import pallas as pl

_F32 = jnp.float32
_PL = 1024


def _dot(a, b):
    return jax.lax.dot_general(a, b, (((1,), (0,)), ((), ())),
                               preferred_element_type=_F32)


def _rollrows(v):
    C, L = v.shape
    sh = jnp.concatenate([jnp.zeros((C, 32), _F32), v[:, :L - 32]], axis=1)
    lane = jax.lax.broadcasted_iota(jnp.int32, (1, L), 1)
    return jnp.where(lane % _PL < 32, jnp.zeros((), _F32), sh)


def _colprev(v):
    C, L = v.shape
    last = v[:, L - _PL:]
    sh = jnp.concatenate([jnp.zeros((C, 1), _F32), last[:, :_PL - 1]],
                         axis=1)
    lane = jax.lax.broadcasted_iota(jnp.int32, (1, _PL), 1)
    first = jnp.where(lane % 32 == 0, jnp.zeros((), _F32), sh)
    if L == _PL:
        return first
    return jnp.concatenate([first, v[:, :L - _PL]], axis=1)


def _chunkmix(u, v):
    C, L = u.shape
    parts = []
    for t in range(L // _PL):
        sl = slice(t * _PL, (t + 1) * _PL)
        parts += [u[:, sl], v[:, sl]]
    return jnp.concatenate(parts, axis=1)


def _head(x, mask, mwt1, mb1, mwt2, mb2, owt, ob, with_mpred):
    if with_mpred:
        h = jax.nn.relu(_dot(mwt1, x) + mb1) * mask
        logit = (_dot(mwt2, h) + mb2) * mask
        this_mask = mask * (logit[1:2] > logit[0:1]).astype(_F32)
    else:
        logit = None
        this_mask = mask
    out = jax.nn.sigmoid(_dot(owt, x) + ob) * this_mask
    return out, logit, this_mask


def _level0_body(x_ref, m_ref, mwt1_ref, mb1_ref, mwt2_ref, mb2_ref,
                 owt_ref, ob_ref, out_ref, logit_ref, tm_ref, mres_ref):
    x = x_ref[0]
    mask = m_ref[0]
    out, logit, this_mask = _head(x, mask, mwt1_ref[...], mb1_ref[...],
                                  mwt2_ref[...], mb2_ref[...],
                                  owt_ref[...], ob_ref[...], True)
    out_ref[0] = out
    logit_ref[0] = logit
    tm_ref[0] = this_mask
    mres_ref[0] = mask - this_mask


def _level0_kernel(x, mask, mw1, mb1, mw2, mb2, ow, ob):
    B, C, S = x.shape
    bs = lambda c: pl.BlockSpec((1, c, S), lambda b: (b, 0, 0))
    full = lambda a: pl.BlockSpec(a.shape, lambda b: (0,) * a.ndim)
    args = (x, mask, mw1.T, mb1[:, None], mw2.T, mb2[:, None],
            ow.T, ob[:, None])
    in_specs = [bs(C), bs(1)] + [full(a) for a in args[2:]]
    out_shape = [jax.ShapeDtypeStruct((B, c, S), _F32) for c in (1, 2, 1, 1)]
    out_specs = [bs(1), bs(2), bs(1), bs(1)]
    fn = pl.pallas_call(_level0_body, grid=(B,), in_specs=in_specs,
                        out_specs=out_specs, out_shape=out_shape)
    return fn(*args)


def _level_body(with_mpred, want_x, Co,
                x_ref, xu_ref, m_ref, skip_ref,
                wa_ref, wb_ref, wc_ref, wd_ref, b_ref, *rest):
    if with_mpred:
        (mwt1_ref, mb1_ref, mwt2_ref, mb2_ref, owt_ref, ob_ref,
         xout_ref, out_ref, logit_ref, tm_ref, mres_ref) = rest
    else:
        mwt1_ref = mb1_ref = mwt2_ref = mb2_ref = None
        owt_ref, ob_ref, out_ref, tm_ref = rest
        xout_ref = logit_ref = mres_ref = None

    S = pl.program_id(1)
    P = x_ref[0, :, 0, 0, :]
    Pu_r = xu_ref[0, :, 0, 0, :]
    Pu = jnp.where(S == 0, _rollrows(Pu_r), Pu_r)
    Pl = _colprev(P)
    Pul = _colprev(Pu)

    A = _dot(wa_ref[...], P)
    Bv = _dot(wb_ref[...], Pu)
    Cv = _dot(wc_ref[...], Pl)
    Dv = _dot(wd_ref[...], Pul)
    p00 = A[:Co] + Bv[:Co] + Cv[:Co] + Dv
    p01 = A[Co:2 * Co] + Bv[Co:2 * Co]
    p10 = A[2 * Co:3 * Co] + Cv[Co:2 * Co]
    p11 = A[3 * Co:]

    m = m_ref[0, :, 0, 0, :]
    mup = _chunkmix(m, m)
    db = b_ref[...]
    for a, (pb0, pb1) in enumerate(((p00, p01), (p10, p11))):
        y = _chunkmix(pb0, pb1) + db
        skip_a = skip_ref[0, :, 0, a, 0, :]
        xk = (jnp.where(y >= 0, y, 0.01 * y) * mup + skip_a * mup) * 0.5
        if want_x:
            xout_ref[0, :, 0, a, 0, :] = xk
        out, logit, this_mask = _head(
            xk, mup,
            None if mwt1_ref is None else mwt1_ref[...],
            None if mb1_ref is None else mb1_ref[...],
            None if mwt2_ref is None else mwt2_ref[...],
            None if mb2_ref is None else mb2_ref[...],
            owt_ref[...], ob_ref[...], with_mpred)
        out_ref[0, :, 0, a, 0, :] = out
        tm_ref[0, :, 0, a, 0, :] = this_mask
        if with_mpred:
            logit_ref[0, :, 0, a, 0, :] = logit
            mres_ref[0, :, 0, a, 0, :] = mup - this_mask


def _level_kernel(x, mask, skip, dw, db, mw1, mb1, mw2, mb2, ow, ob,
                  G, with_mpred, want_x):
    B, Cin = x.shape[0], x.shape[1]
    Co = dw.shape[3]
    L = G * _PL
    grid = (B, G)
    wt = lambda kh, kw: dw[kh, kw].T
    wa = jnp.concatenate([wt(2, 2), wt(2, 1), wt(1, 2), wt(1, 1)], axis=0)
    wb = jnp.concatenate([wt(0, 2), wt(0, 1)], axis=0)
    wc = jnp.concatenate([wt(2, 0), wt(1, 0)], axis=0)
    wd = wt(0, 0)

    xbs = lambda c: pl.BlockSpec((1, c, 1, 1, L),
                                 lambda b, s: (b, 0, s, 0, 0))
    xbs_u = pl.BlockSpec((1, Cin, 1, 1, L),
                         lambda b, s: (b, 0, (s - 1) % G, 0, 0))
    obs = lambda c: pl.BlockSpec((1, c, 1, 2, 1, 2 * L),
                                 lambda b, s: (b, 0, s, 0, 0, 0))
    full = lambda a: pl.BlockSpec(a.shape, lambda b, s: (0,) * a.ndim)

    args = [x, x, mask, skip, wa, wb, wc, wd, db[:, None]]
    in_specs = [xbs(Cin), xbs_u, xbs(1), obs(Co), full(wa), full(wb),
                full(wc), full(wd), full(args[-1])]
    if with_mpred:
        args += [mw1.T, mb1[:, None], mw2.T, mb2[:, None]]
        in_specs += [full(a) for a in args[-4:]]
    args += [ow.T, ob[:, None]]
    in_specs += [full(args[-2]), full(args[-1])]

    oshape = lambda c: jax.ShapeDtypeStruct((B, c, G, 2, 1, 2 * L), _F32)
    out_shape, out_specs = [], []
    if want_x:
        out_shape.append(oshape(Co))
        out_specs.append(obs(Co))
    out_shape.append(oshape(1))
    out_specs.append(obs(1))
    if with_mpred:
        out_shape.append(oshape(2))
        out_specs.append(obs(2))
    out_shape.append(oshape(1))
    out_specs.append(obs(1))
    if with_mpred:
        out_shape.append(oshape(1))
        out_specs.append(obs(1))

    fn = pl.pallas_call(
        functools.partial(_level_body, with_mpred, want_x, Co),
        grid=grid, in_specs=in_specs, out_specs=out_specs,
        out_shape=out_shape)
    outs = fn(*args)
    return [o.reshape(o.shape[0], o.shape[1], 2 * G, 1, 2 * L)
            for o in outs]


def _to_rp(a, D):
    P2 = 2 ** D
    B, C = a.shape[0], a.shape[1]
    a = a.reshape(B, C, 32, P2, 32, P2).transpose(0, 1, 3, 5, 2, 4)
    return a.reshape(B, C, P2, 1, P2 * _PL)


def _from_rp(a, D):
    P2 = 2 ** D
    B, C = a.shape[0], a.shape[1]
    a = a.reshape(B, C, P2, P2, 32, 32).transpose(0, 1, 4, 2, 5, 3)
    return a.reshape(B, C, 32 * P2, 32 * P2)


def kernel(input, imask, skip_1, skip_2, skip_3, dw1, db1, dw2, db2, dw3,
           db3, ow0, ob0, ow1, ob1, ow2, ob2, ow3, ob3, mw1_0, mb1_0,
           mw2_0, mb2_0, mw1_1, mb1_1, mw2_1, mb2_1, mw1_2, mb1_2, mw2_2,
           mb2_2):
    B = input.shape[0]
    out0, l0, tm0, mres0 = _level0_kernel(
        input.reshape(B, 256, _PL), imask.reshape(B, 1, _PL),
        mw1_0, mb1_0, mw2_0, mb2_0, ow0, ob0)
    x1, out1, l1, tm1, mres1 = _level_kernel(
        input.reshape(B, 256, 1, 1, _PL), mres0.reshape(B, 1, 1, 1, _PL),
        _to_rp(skip_1, 1).reshape(B, 128, 1, 2, 1, 2 * _PL),
        dw1, db1, mw1_1, mb1_1, mw2_1, mb2_1, ow1, ob1,
        G=1, with_mpred=True, want_x=True)
    x2, out2, l2, tm2, mres2 = _level_kernel(
        x1, mres1, _to_rp(skip_2, 2).reshape(B, 64, 2, 2, 1, 4 * _PL),
        dw2, db2, mw1_2, mb1_2, mw2_2, mb2_2, ow2, ob2,
        G=2, with_mpred=True, want_x=True)
    out3, tm3 = _level_kernel(
        x2, mres2, _to_rp(skip_3, 3).reshape(B, 32, 4, 2, 1, 8 * _PL),
        dw3, db3, None, None, None, None, ow3, ob3,
        G=4, with_mpred=False, want_x=False)
    sq = lambda a, c: a.reshape(B, c, 32, 32)
    return (sq(out0, 1), _from_rp(out1, 1), _from_rp(out2, 2),
            _from_rp(out3, 3), sq(l0, 2), _from_rp(l1, 1),
            _from_rp(l2, 2), sq(tm0, 1), _from_rp(tm1, 1),
            _from_rp(tm2, 2), _from_rp(tm3, 3))


# speedup vs baseline: 7.7454x; 1.1099x over previous
import functools

import jax
import jax.numpy as jnp
from jax.experimental import pallas as pl

_F32 = jnp.float32
_PL = 1024


def _dot(a, b):
    return jax.lax.dot_general(a, b, (((1,), (0,)), ((), ())),
                               preferred_element_type=_F32)


def _rollrows(v):
    C, L = v.shape
    sh = jnp.concatenate([jnp.zeros((C, 32), _F32), v[:, :L - 32]], axis=1)
    lane = jax.lax.broadcasted_iota(jnp.int32, (1, L), 1)
    return jnp.where(lane % _PL < 32, jnp.zeros((), _F32), sh)


def _colprev(v):
    C, L = v.shape
    last = v[:, L - _PL:]
    sh = jnp.concatenate([jnp.zeros((C, 1), _F32), last[:, :_PL - 1]],
                         axis=1)
    lane = jax.lax.broadcasted_iota(jnp.int32, (1, _PL), 1)
    first = jnp.where(lane % 32 == 0, jnp.zeros((), _F32), sh)
    if L == _PL:
        return first
    return jnp.concatenate([first, v[:, :L - _PL]], axis=1)


def _chunkmix(u, v):
    C, L = u.shape
    parts = []
    for t in range(L // _PL):
        sl = slice(t * _PL, (t + 1) * _PL)
        parts += [u[:, sl], v[:, sl]]
    return jnp.concatenate(parts, axis=1)


def _head(x, mask, mwt1, mb1, mwt2, mb2, owt, ob, with_mpred):
    if with_mpred:
        h = jax.nn.relu(_dot(mwt1, x) + mb1) * mask
        logit = (_dot(mwt2, h) + mb2) * mask
        this_mask = mask * (logit[1:2] > logit[0:1]).astype(_F32)
    else:
        logit = None
        this_mask = mask
    out = jax.nn.sigmoid(_dot(owt, x) + ob) * this_mask
    return out, logit, this_mask


def _level0_body(x_ref, m_ref, mwt1_ref, mb1_ref, mwt2_ref, mb2_ref,
                 owt_ref, ob_ref, out_ref, logit_ref, tm_ref, mres_ref):
    x = x_ref[0]
    mask = m_ref[0]
    out, logit, this_mask = _head(x, mask, mwt1_ref[...], mb1_ref[...],
                                  mwt2_ref[...], mb2_ref[...],
                                  owt_ref[...], ob_ref[...], True)
    out_ref[0] = out
    logit_ref[0] = logit
    tm_ref[0] = this_mask
    mres_ref[0] = mask - this_mask


def _level0_kernel(x, mask, mw1, mb1, mw2, mb2, ow, ob):
    B, C, S = x.shape
    bs = lambda c: pl.BlockSpec((1, c, S), lambda b: (b, 0, 0))
    full = lambda a: pl.BlockSpec(a.shape, lambda b: (0,) * a.ndim)
    args = (x, mask, mw1.T, mb1[:, None], mw2.T, mb2[:, None],
            ow.T, ob[:, None])
    in_specs = [bs(C), bs(1)] + [full(a) for a in args[2:]]
    out_shape = [jax.ShapeDtypeStruct((B, c, S), _F32) for c in (1, 2, 1, 1)]
    out_specs = [bs(1), bs(2), bs(1), bs(1)]
    fn = pl.pallas_call(_level0_body, grid=(B,), in_specs=in_specs,
                        out_specs=out_specs, out_shape=out_shape)
    return fn(*args)


def _level_body(with_mpred, want_x, Co,
                x_ref, xu_ref, m_ref, skip_ref,
                wa_ref, wb_ref, wc_ref, wd_ref, b_ref, *rest):
    if with_mpred:
        (mwt1_ref, mb1_ref, mwt2_ref, mb2_ref, owt_ref, ob_ref,
         xout_ref, out_ref, logit_ref, tm_ref, mres_ref) = rest
    else:
        mwt1_ref = mb1_ref = mwt2_ref = mb2_ref = None
        owt_ref, ob_ref, out_ref, tm_ref = rest
        xout_ref = logit_ref = mres_ref = None

    S = pl.program_id(1)
    P = x_ref[0, :, 0, 0, :]
    Pu_r = xu_ref[0, :, 0, 0, :]
    Pu = jnp.where(S == 0, _rollrows(Pu_r), Pu_r)
    Pl = _colprev(P)
    Pul = _colprev(Pu)

    A = _dot(wa_ref[...], P)
    Bv = _dot(wb_ref[...], Pu)
    Cv = _dot(wc_ref[...], Pl)
    Dv = _dot(wd_ref[...], Pul)
    p00 = A[:Co] + Bv[:Co] + Cv[:Co] + Dv
    p01 = A[Co:2 * Co] + Bv[Co:2 * Co]
    p10 = A[2 * Co:3 * Co] + Cv[Co:2 * Co]
    p11 = A[3 * Co:]

    m = m_ref[0, :, 0, 0, :]
    mup = _chunkmix(m, m)
    db = b_ref[...]
    for a, (pb0, pb1) in enumerate(((p00, p01), (p10, p11))):
        y = _chunkmix(pb0, pb1) + db
        skip_a = skip_ref[0, :, 0, a, 0, :]
        xk = (jnp.where(y >= 0, y, 0.01 * y) * mup + skip_a * mup) * 0.5
        if want_x:
            xout_ref[0, :, 0, a, 0, :] = xk
        out, logit, this_mask = _head(
            xk, mup,
            None if mwt1_ref is None else mwt1_ref[...],
            None if mb1_ref is None else mb1_ref[...],
            None if mwt2_ref is None else mwt2_ref[...],
            None if mb2_ref is None else mb2_ref[...],
            owt_ref[...], ob_ref[...], with_mpred)
        out_ref[0, :, 0, a, 0, :] = out
        tm_ref[0, :, 0, a, 0, :] = this_mask
        if with_mpred:
            logit_ref[0, :, 0, a, 0, :] = logit
            mres_ref[0, :, 0, a, 0, :] = mup - this_mask


def _level_kernel(x, mask, skip, dw, db, mw1, mb1, mw2, mb2, ow, ob,
                  G, with_mpred, want_x):
    B, Cin = x.shape[0], x.shape[1]
    Co = dw.shape[3]
    L = G * _PL
    grid = (B, G)
    wt = lambda kh, kw: dw[kh, kw].T
    wa = jnp.concatenate([wt(2, 2), wt(2, 1), wt(1, 2), wt(1, 1)], axis=0)
    wb = jnp.concatenate([wt(0, 2), wt(0, 1)], axis=0)
    wc = jnp.concatenate([wt(2, 0), wt(1, 0)], axis=0)
    wd = wt(0, 0)

    xbs = lambda c: pl.BlockSpec((1, c, 1, 1, L),
                                 lambda b, s: (b, 0, s, 0, 0))
    xbs_u = pl.BlockSpec((1, Cin, 1, 1, L),
                         lambda b, s: (b, 0, (s - 1) % G, 0, 0))
    obs = lambda c: pl.BlockSpec((1, c, 1, 2, 1, 2 * L),
                                 lambda b, s: (b, 0, s, 0, 0, 0))
    full = lambda a: pl.BlockSpec(a.shape, lambda b, s: (0,) * a.ndim)

    args = [x, x, mask, skip, wa, wb, wc, wd, db[:, None]]
    in_specs = [xbs(Cin), xbs_u, xbs(1), obs(Co), full(wa), full(wb),
                full(wc), full(wd), full(args[-1])]
    if with_mpred:
        args += [mw1.T, mb1[:, None], mw2.T, mb2[:, None]]
        in_specs += [full(a) for a in args[-4:]]
    args += [ow.T, ob[:, None]]
    in_specs += [full(args[-2]), full(args[-1])]

    oshape = lambda c: jax.ShapeDtypeStruct((B, c, G, 2, 1, 2 * L), _F32)
    out_shape, out_specs = [], []
    if want_x:
        out_shape.append(oshape(Co))
        out_specs.append(obs(Co))
    out_shape.append(oshape(1))
    out_specs.append(obs(1))
    if with_mpred:
        out_shape.append(oshape(2))
        out_specs.append(obs(2))
    out_shape.append(oshape(1))
    out_specs.append(obs(1))
    if with_mpred:
        out_shape.append(oshape(1))
        out_specs.append(obs(1))

    fn = pl.pallas_call(
        functools.partial(_level_body, with_mpred, want_x, Co),
        grid=grid, in_specs=in_specs, out_specs=out_specs,
        out_shape=out_shape)
    outs = fn(*args)
    return [o.reshape(o.shape[0], o.shape[1], 2 * G, 1, 2 * L)
            for o in outs]


def _to_rp(a, D):
    P2 = 2 ** D
    B, C = a.shape[0], a.shape[1]
    a = a.reshape(B, C, 32, P2, 32 * P2).transpose(0, 1, 3, 2, 4)
    a = a.reshape(B, C, P2, 32, 32, P2).transpose(0, 1, 2, 5, 3, 4)
    return a.reshape(B, C, P2, 1, P2 * _PL)


def _from_rp(a, D):
    P2 = 2 ** D
    B, C = a.shape[0], a.shape[1]
    a = a.reshape(B, C, P2, P2, 32, 32).transpose(0, 1, 2, 4, 5, 3)
    a = a.reshape(B, C, P2, 32, 32 * P2).transpose(0, 1, 3, 2, 4)
    return a.reshape(B, C, 32 * P2, 32 * P2)


def kernel(input, imask, skip_1, skip_2, skip_3, dw1, db1, dw2, db2, dw3,
           db3, ow0, ob0, ow1, ob1, ow2, ob2, ow3, ob3, mw1_0, mb1_0,
           mw2_0, mb2_0, mw1_1, mb1_1, mw2_1, mb2_1, mw1_2, mb1_2, mw2_2,
           mb2_2):
    B = input.shape[0]
    out0, l0, tm0, mres0 = _level0_kernel(
        input.reshape(B, 256, _PL), imask.reshape(B, 1, _PL),
        mw1_0, mb1_0, mw2_0, mb2_0, ow0, ob0)
    x1, out1, l1, tm1, mres1 = _level_kernel(
        input.reshape(B, 256, 1, 1, _PL), mres0.reshape(B, 1, 1, 1, _PL),
        _to_rp(skip_1, 1).reshape(B, 128, 1, 2, 1, 2 * _PL),
        dw1, db1, mw1_1, mb1_1, mw2_1, mb2_1, ow1, ob1,
        G=1, with_mpred=True, want_x=True)
    x2, out2, l2, tm2, mres2 = _level_kernel(
        x1, mres1, _to_rp(skip_2, 2).reshape(B, 64, 2, 2, 1, 4 * _PL),
        dw2, db2, mw1_2, mb1_2, mw2_2, mb2_2, ow2, ob2,
        G=2, with_mpred=True, want_x=True)
    out3, tm3 = _level_kernel(
        x2, mres2, _to_rp(skip_3, 3).reshape(B, 32, 4, 2, 1, 8 * _PL),
        dw3, db3, None, None, None, None, ow3, ob3,
        G=4, with_mpred=False, want_x=False)
    sq = lambda a, c: a.reshape(B, c, 32, 32)
    return (sq(out0, 1), _from_rp(out1, 1), _from_rp(out2, 2),
            _from_rp(out3, 3), sq(l0, 2), _from_rp(l1, 1),
            _from_rp(l2, 2), sq(tm0, 1), _from_rp(tm1, 1),
            _from_rp(tm2, 2), _from_rp(tm3, 3))


# speedup vs baseline: 7.7543x; 1.0011x over previous
import functools

import jax
import jax.numpy as jnp
from jax.experimental import pallas as pl
from jax.experimental.pallas import tpu as pltpu

_F32 = jnp.float32
_PL = 1024


def _dot(a, b):
    return jax.lax.dot_general(a, b, (((1,), (0,)), ((), ())),
                               preferred_element_type=_F32)


def _rollrows(v):
    C, L = v.shape
    sh = jnp.concatenate([jnp.zeros((C, 32), _F32), v[:, :L - 32]], axis=1)
    lane = jax.lax.broadcasted_iota(jnp.int32, (1, L), 1)
    return jnp.where(lane % _PL < 32, jnp.zeros((), _F32), sh)


def _colprev(v):
    C, L = v.shape
    last = v[:, L - _PL:]
    sh = jnp.concatenate([jnp.zeros((C, 1), _F32), last[:, :_PL - 1]],
                         axis=1)
    lane = jax.lax.broadcasted_iota(jnp.int32, (1, _PL), 1)
    first = jnp.where(lane % 32 == 0, jnp.zeros((), _F32), sh)
    if L == _PL:
        return first
    return jnp.concatenate([first, v[:, :L - _PL]], axis=1)


def _chunkmix(u, v):
    C, L = u.shape
    parts = []
    for t in range(L // _PL):
        sl = slice(t * _PL, (t + 1) * _PL)
        parts += [u[:, sl], v[:, sl]]
    return jnp.concatenate(parts, axis=1)


def _head(x, mask, mwt1, mb1, mwt2, mb2, owt, ob, with_mpred):
    if with_mpred:
        h = jax.nn.relu(_dot(mwt1, x) + mb1) * mask
        logit = (_dot(mwt2, h) + mb2) * mask
        this_mask = mask * (logit[1:2] > logit[0:1]).astype(_F32)
    else:
        logit = None
        this_mask = mask
    out = jax.nn.sigmoid(_dot(owt, x) + ob) * this_mask
    return out, logit, this_mask


def _level0_body(x_ref, m_ref, mwt1_ref, mb1_ref, mwt2_ref, mb2_ref,
                 owt_ref, ob_ref, out_ref, logit_ref, tm_ref, mres_ref):
    x = x_ref[0]
    mask = m_ref[0]
    out, logit, this_mask = _head(x, mask, mwt1_ref[...], mb1_ref[...],
                                  mwt2_ref[...], mb2_ref[...],
                                  owt_ref[...], ob_ref[...], True)
    out_ref[0] = out
    logit_ref[0] = logit
    tm_ref[0] = this_mask
    mres_ref[0] = mask - this_mask


def _level0_kernel(x, mask, mw1, mb1, mw2, mb2, ow, ob):
    B, C, S = x.shape
    bs = lambda c: pl.BlockSpec((1, c, S), lambda b: (b, 0, 0))
    full = lambda a: pl.BlockSpec(a.shape, lambda b: (0,) * a.ndim)
    args = (x, mask, mw1.T, mb1[:, None], mw2.T, mb2[:, None],
            ow.T, ob[:, None])
    in_specs = [bs(C), bs(1)] + [full(a) for a in args[2:]]
    out_shape = [jax.ShapeDtypeStruct((B, c, S), _F32) for c in (1, 2, 1, 1)]
    out_specs = [bs(1), bs(2), bs(1), bs(1)]
    fn = pl.pallas_call(_level0_body, grid=(B,), in_specs=in_specs,
                        out_specs=out_specs, out_shape=out_shape)
    return fn(*args)


def _level_body(with_mpred, want_x, Co, G, x_ref, *more):
    if G == 1:
        xu_ref = None
        (m_ref, skip_ref, wa_ref, wb_ref, wc_ref, wd_ref, b_ref,
         *rest) = more
    else:
        (xu_ref, m_ref, skip_ref, wa_ref, wb_ref, wc_ref, wd_ref, b_ref,
         *rest) = more
    if with_mpred:
        (mwt1_ref, mb1_ref, mwt2_ref, mb2_ref, owt_ref, ob_ref,
         xout_ref, out_ref, logit_ref, tm_ref, mres_ref) = rest
    else:
        mwt1_ref = mb1_ref = mwt2_ref = mb2_ref = None
        owt_ref, ob_ref, out_ref, tm_ref = rest
        xout_ref = logit_ref = mres_ref = None

    S = pl.program_id(1)
    P = x_ref[0, :, 0, 0, :]
    if G == 1:
        Pu = _rollrows(P)
    else:
        Pu_r = xu_ref[0, :, 0, 0, :]
        Pu = jnp.where(S == 0, _rollrows(Pu_r), Pu_r)
    Pl = _colprev(P)
    Pul = _colprev(Pu)

    A = _dot(wa_ref[...], P)
    Bv = _dot(wb_ref[...], Pu)
    Cv = _dot(wc_ref[...], Pl)
    Dv = _dot(wd_ref[...], Pul)
    p00 = A[:Co] + Bv[:Co] + Cv[:Co] + Dv
    p01 = A[Co:2 * Co] + Bv[Co:2 * Co]
    p10 = A[2 * Co:3 * Co] + Cv[Co:2 * Co]
    p11 = A[3 * Co:]

    m = m_ref[0, :, 0, 0, :]
    mup = _chunkmix(m, m)
    db = b_ref[...]
    for a, (pb0, pb1) in enumerate(((p00, p01), (p10, p11))):
        y = _chunkmix(pb0, pb1) + db
        skip_a = skip_ref[0, :, 0, a, 0, :]
        xk = (jnp.where(y >= 0, y, 0.01 * y) * mup + skip_a * mup) * 0.5
        if want_x:
            xout_ref[0, :, 0, a, 0, :] = xk
        out, logit, this_mask = _head(
            xk, mup,
            None if mwt1_ref is None else mwt1_ref[...],
            None if mb1_ref is None else mb1_ref[...],
            None if mwt2_ref is None else mwt2_ref[...],
            None if mb2_ref is None else mb2_ref[...],
            owt_ref[...], ob_ref[...], with_mpred)
        out_ref[0, :, 0, a, 0, :] = out
        tm_ref[0, :, 0, a, 0, :] = this_mask
        if with_mpred:
            logit_ref[0, :, 0, a, 0, :] = logit
            mres_ref[0, :, 0, a, 0, :] = mup - this_mask


def _level_kernel(x, mask, skip, dw, db, mw1, mb1, mw2, mb2, ow, ob,
                  G, with_mpred, want_x):
    B, Cin = x.shape[0], x.shape[1]
    Co = dw.shape[3]
    L = G * _PL
    grid = (B, G)
    wt = lambda kh, kw: dw[kh, kw].T
    wa = jnp.concatenate([wt(2, 2), wt(2, 1), wt(1, 2), wt(1, 1)], axis=0)
    wb = jnp.concatenate([wt(0, 2), wt(0, 1)], axis=0)
    wc = jnp.concatenate([wt(2, 0), wt(1, 0)], axis=0)
    wd = wt(0, 0)

    xbs = lambda c: pl.BlockSpec((1, c, 1, 1, L),
                                 lambda b, s: (b, 0, s, 0, 0))
    xbs_u = pl.BlockSpec((1, Cin, 1, 1, L),
                         lambda b, s: (b, 0, (s - 1) % G, 0, 0))
    obs = lambda c: pl.BlockSpec((1, c, 1, 2, 1, 2 * L),
                                 lambda b, s: (b, 0, s, 0, 0, 0))
    full = lambda a: pl.BlockSpec(a.shape, lambda b, s: (0,) * a.ndim)

    if G == 1:
        args = [x, mask, skip, wa, wb, wc, wd, db[:, None]]
        in_specs = [xbs(Cin), xbs(1), obs(Co), full(wa), full(wb),
                    full(wc), full(wd), full(args[-1])]
    else:
        args = [x, x, mask, skip, wa, wb, wc, wd, db[:, None]]
        in_specs = [xbs(Cin), xbs_u, xbs(1), obs(Co), full(wa), full(wb),
                    full(wc), full(wd), full(args[-1])]
    if with_mpred:
        args += [mw1.T, mb1[:, None], mw2.T, mb2[:, None]]
        in_specs += [full(a) for a in args[-4:]]
    args += [ow.T, ob[:, None]]
    in_specs += [full(args[-2]), full(args[-1])]

    oshape = lambda c: jax.ShapeDtypeStruct((B, c, G, 2, 1, 2 * L), _F32)
    out_shape, out_specs = [], []
    if want_x:
        out_shape.append(oshape(Co))
        out_specs.append(obs(Co))
    out_shape.append(oshape(1))
    out_specs.append(obs(1))
    if with_mpred:
        out_shape.append(oshape(2))
        out_specs.append(obs(2))
    out_shape.append(oshape(1))
    out_specs.append(obs(1))
    if with_mpred:
        out_shape.append(oshape(1))
        out_specs.append(obs(1))

    fn = pl.pallas_call(
        functools.partial(_level_body, with_mpred, want_x, Co, G),
        grid=grid, in_specs=in_specs, out_specs=out_specs,
        out_shape=out_shape,
        compiler_params=pltpu.CompilerParams(
            dimension_semantics=("parallel", "arbitrary")))
    outs = fn(*args)
    return [o.reshape(o.shape[0], o.shape[1], 2 * G, 1, 2 * L)
            for o in outs]


def _to_rp(a, D):
    P2 = 2 ** D
    B, C = a.shape[0], a.shape[1]
    a = a.reshape(B, C, 32, P2, 32 * P2).transpose(0, 1, 3, 2, 4)
    a = a.reshape(B, C, P2, 32, 32, P2).transpose(0, 1, 2, 5, 3, 4)
    return a.reshape(B, C, P2, 1, P2 * _PL)


def _from_rp(a, D):
    P2 = 2 ** D
    B, C = a.shape[0], a.shape[1]
    a = a.reshape(B, C, P2, P2, 32, 32).transpose(0, 1, 2, 4, 5, 3)
    a = a.reshape(B, C, P2, 32, 32 * P2).transpose(0, 1, 3, 2, 4)
    return a.reshape(B, C, 32 * P2, 32 * P2)


def kernel(input, imask, skip_1, skip_2, skip_3, dw1, db1, dw2, db2, dw3,
           db3, ow0, ob0, ow1, ob1, ow2, ob2, ow3, ob3, mw1_0, mb1_0,
           mw2_0, mb2_0, mw1_1, mb1_1, mw2_1, mb2_1, mw1_2, mb1_2, mw2_2,
           mb2_2):
    B = input.shape[0]
    out0, l0, tm0, mres0 = _level0_kernel(
        input.reshape(B, 256, _PL), imask.reshape(B, 1, _PL),
        mw1_0, mb1_0, mw2_0, mb2_0, ow0, ob0)
    x1, out1, l1, tm1, mres1 = _level_kernel(
        input.reshape(B, 256, 1, 1, _PL), mres0.reshape(B, 1, 1, 1, _PL),
        _to_rp(skip_1, 1).reshape(B, 128, 1, 2, 1, 2 * _PL),
        dw1, db1, mw1_1, mb1_1, mw2_1, mb2_1, ow1, ob1,
        G=1, with_mpred=True, want_x=True)
    x2, out2, l2, tm2, mres2 = _level_kernel(
        x1, mres1, _to_rp(skip_2, 2).reshape(B, 64, 2, 2, 1, 4 * _PL),
        dw2, db2, mw1_2, mb1_2, mw2_2, mb2_2, ow2, ob2,
        G=2, with_mpred=True, want_x=True)
    out3, tm3 = _level_kernel(
        x2, mres2, _to_rp(skip_3, 3).reshape(B, 32, 4, 2, 1, 8 * _PL),
        dw3, db3, None, None, None, None, ow3, ob3,
        G=4, with_mpred=False, want_x=False)
    sq = lambda a, c: a.reshape(B, c, 32, 32)
    return (sq(out0, 1), _from_rp(out1, 1), _from_rp(out2, 2),
            _from_rp(out3, 3), sq(l0, 2), _from_rp(l1, 1),
            _from_rp(l2, 2), sq(tm0, 1), _from_rp(tm1, 1),
            _from_rp(tm2, 2), _from_rp(tm3, 3))


# speedup vs baseline: 11.6031x; 1.4963x over previous
import functools

import jax
import jax.numpy as jnp
from jax.experimental import pallas as pl
from jax.experimental.pallas import tpu as pltpu

_F32 = jnp.float32
_PL = 1024


def _dot(a, b):
    return jax.lax.dot_general(a, b, (((1,), (0,)), ((), ())),
                               preferred_element_type=_F32)


def _rollrows(v):
    C, L = v.shape
    sh = jnp.concatenate([jnp.zeros((C, 32), _F32), v[:, :L - 32]], axis=1)
    lane = jax.lax.broadcasted_iota(jnp.int32, (1, L), 1)
    return jnp.where(lane % _PL < 32, jnp.zeros((), _F32), sh)


def _colprev(v):
    C, L = v.shape
    last = v[:, L - _PL:]
    sh = jnp.concatenate([jnp.zeros((C, 1), _F32), last[:, :_PL - 1]],
                         axis=1)
    lane = jax.lax.broadcasted_iota(jnp.int32, (1, _PL), 1)
    first = jnp.where(lane % 32 == 0, jnp.zeros((), _F32), sh)
    if L == _PL:
        return first
    return jnp.concatenate([first, v[:, :L - _PL]], axis=1)


def _chunkmix(u, v):
    C, L = u.shape
    parts = []
    for t in range(L // _PL):
        sl = slice(t * _PL, (t + 1) * _PL)
        parts += [u[:, sl], v[:, sl]]
    return jnp.concatenate(parts, axis=1)


def _head(x, mask, mwt1, mb1, mwt2, mb2, owt, ob, with_mpred):
    if with_mpred:
        h = jax.nn.relu(_dot(mwt1, x) + mb1) * mask
        logit = (_dot(mwt2, h) + mb2) * mask
        this_mask = mask * (logit[1:2] > logit[0:1]).astype(_F32)
    else:
        logit = None
        this_mask = mask
    out = jax.nn.sigmoid(_dot(owt, x) + ob) * this_mask
    return out, logit, this_mask


def _level0_body(x_ref, m_ref, mwt1_ref, mb1_ref, mwt2_ref, mb2_ref,
                 owt_ref, ob_ref, out_ref, logit_ref, tm_ref, mres_ref):
    x = x_ref[0]
    mask = m_ref[0]
    out, logit, this_mask = _head(x, mask, mwt1_ref[...], mb1_ref[...],
                                  mwt2_ref[...], mb2_ref[...],
                                  owt_ref[...], ob_ref[...], True)
    out_ref[0] = out
    logit_ref[0] = logit
    tm_ref[0] = this_mask
    mres_ref[0] = mask - this_mask


def _level0_kernel(x, mask, mw1, mb1, mw2, mb2, ow, ob):
    B, C, S = x.shape
    bs = lambda c: pl.BlockSpec((1, c, S), lambda b: (b, 0, 0))
    full = lambda a: pl.BlockSpec(a.shape, lambda b: (0,) * a.ndim)
    args = (x, mask, mw1.T, mb1[:, None], mw2.T, mb2[:, None],
            ow.T, ob[:, None])
    in_specs = [bs(C), bs(1)] + [full(a) for a in args[2:]]
    out_shape = [jax.ShapeDtypeStruct((B, c, S), _F32) for c in (1, 2, 1, 1)]
    out_specs = [bs(1), bs(2), bs(1), bs(1)]
    fn = pl.pallas_call(_level0_body, grid=(B,), in_specs=in_specs,
                        out_specs=out_specs, out_shape=out_shape)
    return fn(*args)


def _proj_body(x_ref, w_ref, o_ref):
    v = x_ref[0]
    o_ref[0] = jnp.sum(v * w_ref[...], axis=0, keepdims=True)


def _proj_kernel(skip, ow):
    B, C, H, W = skip.shape
    R = 32
    grid = (B, H // R)
    in_specs = [pl.BlockSpec((1, C, R, W), lambda b, i: (b, 0, i, 0)),
                pl.BlockSpec((C, 1, 1), lambda b, i: (0, 0, 0))]
    out_spec = pl.BlockSpec((1, 1, R, W), lambda b, i: (b, 0, i, 0))
    fn = pl.pallas_call(
        _proj_body, grid=grid, in_specs=in_specs, out_specs=out_spec,
        out_shape=jax.ShapeDtypeStruct((B, 1, H, W), _F32),
        compiler_params=pltpu.CompilerParams(
            dimension_semantics=("parallel", "parallel")))
    return fn(skip, ow.reshape(C, 1, 1))


def _level_body(with_mpred, want_x, Co, G, x_ref, *more):
    if G == 1:
        xu_ref = None
        (m_ref, skip_ref, wa_ref, wb_ref, wc_ref, wd_ref, b_ref,
         *rest) = more
    else:
        (xu_ref, m_ref, skip_ref, wa_ref, wb_ref, wc_ref, wd_ref, b_ref,
         *rest) = more
    if with_mpred:
        (mwt1_ref, mb1_ref, mwt2_ref, mb2_ref, owt_ref, ob_ref,
         xout_ref, out_ref, logit_ref, tm_ref, mres_ref) = rest
    else:
        mwt1_ref = mb1_ref = mwt2_ref = mb2_ref = None
        owt_ref, ob_ref, out_ref, tm_ref = rest
        xout_ref = logit_ref = mres_ref = None

    S = pl.program_id(1)
    P = x_ref[0, :, 0, 0, :]
    if G == 1:
        Pu = _rollrows(P)
    else:
        Pu_r = xu_ref[0, :, 0, 0, :]
        Pu = jnp.where(S == 0, _rollrows(Pu_r), Pu_r)
    Pl = _colprev(P)
    Pul = _colprev(Pu)

    A = _dot(wa_ref[...], P)
    Bv = _dot(wb_ref[...], Pu)
    Cv = _dot(wc_ref[...], Pl)
    Dv = _dot(wd_ref[...], Pul)
    p00 = A[:Co] + Bv[:Co] + Cv[:Co] + Dv
    p01 = A[Co:2 * Co] + Bv[Co:2 * Co]
    p10 = A[2 * Co:3 * Co] + Cv[Co:2 * Co]
    p11 = A[3 * Co:]

    m = m_ref[0, :, 0, 0, :]
    mup = _chunkmix(m, m)
    db = b_ref[...]
    for a, (pb0, pb1) in enumerate(((p00, p01), (p10, p11))):
        y = _chunkmix(pb0, pb1) + db
        skip_a = skip_ref[0, :, 0, a, 0, :]
        if with_mpred:
            xk = (jnp.where(y >= 0, y, 0.01 * y) * mup
                  + skip_a * mup) * 0.5
            if want_x:
                xout_ref[0, :, 0, a, 0, :] = xk
            out, logit, this_mask = _head(
                xk, mup, mwt1_ref[...], mb1_ref[...], mwt2_ref[...],
                mb2_ref[...], owt_ref[...], ob_ref[...], True)
            out_ref[0, :, 0, a, 0, :] = out
            tm_ref[0, :, 0, a, 0, :] = this_mask
            logit_ref[0, :, 0, a, 0, :] = logit
            mres_ref[0, :, 0, a, 0, :] = mup - this_mask
        else:
            z = jnp.where(y >= 0, y, 0.01 * y)
            g = _dot(owt_ref[...], z)
            v = (g + skip_a) * (0.5 * mup) + ob_ref[...]
            out_ref[0, :, 0, a, 0, :] = jax.nn.sigmoid(v) * mup
            tm_ref[0, :, 0, a, 0, :] = mup


def _level_kernel(x, mask, skip, dw, db, mw1, mb1, mw2, mb2, ow, ob,
                  G, with_mpred, want_x):
    B, Cin = x.shape[0], x.shape[1]
    Co = dw.shape[3]
    L = G * _PL
    grid = (B, G)
    wt = lambda kh, kw: dw[kh, kw].T
    wa = jnp.concatenate([wt(2, 2), wt(2, 1), wt(1, 2), wt(1, 1)], axis=0)
    wb = jnp.concatenate([wt(0, 2), wt(0, 1)], axis=0)
    wc = jnp.concatenate([wt(2, 0), wt(1, 0)], axis=0)
    wd = wt(0, 0)

    xbs = lambda c: pl.BlockSpec((1, c, 1, 1, L),
                                 lambda b, s: (b, 0, s, 0, 0))
    xbs_u = pl.BlockSpec((1, Cin, 1, 1, L),
                         lambda b, s: (b, 0, (s - 1) % G, 0, 0))
    obs = lambda c: pl.BlockSpec((1, c, 1, 2, 1, 2 * L),
                                 lambda b, s: (b, 0, s, 0, 0, 0))
    full = lambda a: pl.BlockSpec(a.shape, lambda b, s: (0,) * a.ndim)

    skip_c = skip.shape[1]
    if G == 1:
        args = [x, mask, skip, wa, wb, wc, wd, db[:, None]]
        in_specs = [xbs(Cin), xbs(1), obs(skip_c), full(wa), full(wb),
                    full(wc), full(wd), full(args[-1])]
    else:
        args = [x, x, mask, skip, wa, wb, wc, wd, db[:, None]]
        in_specs = [xbs(Cin), xbs_u, xbs(1), obs(skip_c), full(wa),
                    full(wb), full(wc), full(wd), full(args[-1])]
    if with_mpred:
        args += [mw1.T, mb1[:, None], mw2.T, mb2[:, None]]
        in_specs += [full(a) for a in args[-4:]]
    args += [ow.T, ob[:, None]]
    in_specs += [full(args[-2]), full(args[-1])]

    oshape = lambda c: jax.ShapeDtypeStruct((B, c, G, 2, 1, 2 * L), _F32)
    out_shape, out_specs = [], []
    if want_x:
        out_shape.append(oshape(Co))
        out_specs.append(obs(Co))
    out_shape.append(oshape(1))
    out_specs.append(obs(1))
    if with_mpred:
        out_shape.append(oshape(2))
        out_specs.append(obs(2))
    out_shape.append(oshape(1))
    out_specs.append(obs(1))
    if with_mpred:
        out_shape.append(oshape(1))
        out_specs.append(obs(1))

    fn = pl.pallas_call(
        functools.partial(_level_body, with_mpred, want_x, Co, G),
        grid=grid, in_specs=in_specs, out_specs=out_specs,
        out_shape=out_shape,
        compiler_params=pltpu.CompilerParams(
            dimension_semantics=("parallel", "arbitrary")))
    outs = fn(*args)
    return [o.reshape(o.shape[0], o.shape[1], 2 * G, 1, 2 * L)
            for o in outs]


def _to_rp(a, D):
    P2 = 2 ** D
    B, C = a.shape[0], a.shape[1]
    a = a.reshape(B, C, 32, P2, 32 * P2).transpose(0, 1, 3, 2, 4)
    a = a.reshape(B, C, P2, 32, 32, P2).transpose(0, 1, 2, 5, 3, 4)
    return a.reshape(B, C, P2, 1, P2 * _PL)


def _from_rp(a, D):
    P2 = 2 ** D
    B, C = a.shape[0], a.shape[1]
    a = a.reshape(B, C, P2, P2, 32, 32).transpose(0, 1, 2, 4, 5, 3)
    a = a.reshape(B, C, P2, 32, 32 * P2).transpose(0, 1, 3, 2, 4)
    return a.reshape(B, C, 32 * P2, 32 * P2)


def kernel(input, imask, skip_1, skip_2, skip_3, dw1, db1, dw2, db2, dw3,
           db3, ow0, ob0, ow1, ob1, ow2, ob2, ow3, ob3, mw1_0, mb1_0,
           mw2_0, mb2_0, mw1_1, mb1_1, mw2_1, mb2_1, mw1_2, mb1_2, mw2_2,
           mb2_2):
    B = input.shape[0]
    out0, l0, tm0, mres0 = _level0_kernel(
        input.reshape(B, 256, _PL), imask.reshape(B, 1, _PL),
        mw1_0, mb1_0, mw2_0, mb2_0, ow0, ob0)
    x1, out1, l1, tm1, mres1 = _level_kernel(
        input.reshape(B, 256, 1, 1, _PL), mres0.reshape(B, 1, 1, 1, _PL),
        _to_rp(skip_1, 1).reshape(B, 128, 1, 2, 1, 2 * _PL),
        dw1, db1, mw1_1, mb1_1, mw2_1, mb2_1, ow1, ob1,
        G=1, with_mpred=True, want_x=True)
    x2, out2, l2, tm2, mres2 = _level_kernel(
        x1, mres1, _to_rp(skip_2, 2).reshape(B, 64, 2, 2, 1, 4 * _PL),
        dw2, db2, mw1_2, mb1_2, mw2_2, mb2_2, ow2, ob2,
        G=2, with_mpred=True, want_x=True)
    s3 = _proj_kernel(skip_3, ow3)
    out3, tm3 = _level_kernel(
        x2, mres2, _to_rp(s3, 3).reshape(B, 1, 4, 2, 1, 8 * _PL),
        dw3, db3, None, None, None, None, ow3, ob3,
        G=4, with_mpred=False, want_x=False)
    sq = lambda a, c: a.reshape(B, c, 32, 32)
    return (sq(out0, 1), _from_rp(out1, 1), _from_rp(out2, 2),
            _from_rp(out3, 3), sq(l0, 2), _from_rp(l1, 1),
            _from_rp(l2, 2), sq(tm0, 1), _from_rp(tm1, 1),
            _from_rp(tm2, 2), _from_rp(tm3, 3))
